# baseline (speedup 1.0000x reference)
import jax
import jax.numpy as jnp
from jax.experimental import pallas as pl

_BS = 128


def _add_body(x_ref, pos_ref, out_ref):
    out_ref[...] = x_ref[...] + pos_ref[...]


def kernel(x, pos_weight):
    batch, seq_len, d_model = x.shape
    bs = _BS if seq_len % _BS == 0 else seq_len
    grid = (seq_len // bs,)
    return pl.pallas_call(
        _add_body,
        grid=grid,
        in_specs=[
            pl.BlockSpec((batch, bs, d_model), lambda s: (0, s, 0)),
            pl.BlockSpec((bs, d_model), lambda s: (s, 0)),
        ],
        out_specs=pl.BlockSpec((batch, bs, d_model), lambda s: (0, s, 0)),
        out_shape=jax.ShapeDtypeStruct(x.shape, x.dtype),
    )(x, pos_weight[:seq_len])


# speedup vs baseline: 1.0844x; 1.0844x over previous
import jax
import jax.numpy as jnp
from jax.experimental import pallas as pl

_BS = 512


def _add_body(x_ref, pos_ref, out_ref):
    out_ref[...] = x_ref[...] + pos_ref[...]


def kernel(x, pos_weight):
    batch, seq_len, d_model = x.shape
    bs = _BS if seq_len % _BS == 0 else seq_len
    grid = (seq_len // bs,)
    return pl.pallas_call(
        _add_body,
        grid=grid,
        in_specs=[
            pl.BlockSpec((batch, bs, d_model), lambda s: (0, s, 0)),
            pl.BlockSpec((bs, d_model), lambda s: (s, 0)),
        ],
        out_specs=pl.BlockSpec((batch, bs, d_model), lambda s: (0, s, 0)),
        out_shape=jax.ShapeDtypeStruct(x.shape, x.dtype),
    )(x, pos_weight[:seq_len])
